# baseline (speedup 1.0000x reference)
import functools

import jax
import jax.numpy as jnp
import numpy as np
from jax import lax
from jax.experimental import pallas as pl
from jax.experimental.pallas import tpu as pltpu
from jax.experimental.pallas import tpu_sc as plsc

N_NODES = 10000
N_EDGES = 320000
D_FEAT = 128
D_EDGE = 16
OUT_DIM = 128
PK = OUT_DIM // 2

NC = 2
NS = 16
NW = NC * NS
EW = N_EDGES // NW
C = 80
NCHUNK = EW // C
UNROLL = 5
NSUPER = NCHUNK // UNROLL


def _leaky(x):
    return jnp.where(x >= 0, x, 0.01 * x)


def _node_proj_body(atom_ref, w_ref, out_ref):
    out_ref[...] = jnp.dot(atom_ref[...], w_ref[...],
                           preferred_element_type=jnp.float32,
                           precision=lax.Precision.HIGHEST)


def _node_proj(atom_feats, w1b):
    return pl.pallas_call(
        _node_proj_body,
        out_shape=jax.ShapeDtypeStruct((N_NODES, OUT_DIM), jnp.float32),
    )(atom_feats, w1b)


def _pack_bf16_pairs(aw):
    lo = lax.bitcast_convert_type(aw[:, :PK].astype(jnp.bfloat16), jnp.uint16)
    hi = lax.bitcast_convert_type(aw[:, PK:].astype(jnp.bfloat16), jnp.uint16)
    packed = (hi.astype(jnp.uint32) << 16) | lo.astype(jnp.uint32)
    return lax.bitcast_convert_type(packed, jnp.int32)


@functools.partial(
    pl.kernel,
    mesh=plsc.VectorSubcoreMesh(core_axis_name="c", subcore_axis_name="s"),
    out_type=(jax.ShapeDtypeStruct((N_EDGES, PK), jnp.int32),
              jax.ShapeDtypeStruct((N_EDGES, PK), jnp.int32)),
    scratch_types=[
        pltpu.VMEM((EW,), jnp.int32),
        pltpu.VMEM((EW,), jnp.int32),
        pltpu.VMEM((2 * UNROLL, C, PK), jnp.int32),
        pltpu.SemaphoreType.DMA,
        pltpu.SemaphoreType.DMA,
    ],
    compiler_params=pltpu.CompilerParams(use_tc_tiling_on_sc=False),
)
def _gather_rows(table, src_hbm, dst_hbm, osrc_hbm, odst_hbm,
                 idx_s, idx_d, rows, sem_g, sem_w):
    wid = lax.axis_index("s") * NC + lax.axis_index("c")
    base = wid * EW
    pltpu.sync_copy(src_hbm.at[pl.ds(base, EW)], idx_s)
    pltpu.sync_copy(dst_hbm.at[pl.ds(base, EW)], idx_d)

    def super_chunk(t, carry):
        j0 = t * UNROLL
        gathers = []
        for u in range(UNROLL):
            sl = pl.ds((j0 + u) * C, C)
            c1 = pltpu.async_copy(table.at[idx_s.at[sl]], rows.at[2 * u], sem_g)
            c2 = pltpu.async_copy(table.at[idx_d.at[sl]], rows.at[2 * u + 1],
                                  sem_g)
            gathers.append((c1, c2))
        writebacks = []
        for u in range(UNROLL):
            c1, c2 = gathers[u]
            c1.wait()
            c2.wait()
            off = base + (j0 + u) * C
            writebacks.append(pltpu.async_copy(
                rows.at[2 * u], osrc_hbm.at[pl.ds(off, C)], sem_w))
            writebacks.append(pltpu.async_copy(
                rows.at[2 * u + 1], odst_hbm.at[pl.ds(off, C)], sem_w))
        for wb in writebacks:
            wb.wait()
        return carry

    lax.fori_loop(0, NSUPER, super_chunk, 0)


BE = 2000

_MHI32 = np.int32(-65536)


def _unpack_block(p):
    lo = lax.bitcast_convert_type(p << 16, jnp.float32)
    hi = lax.bitcast_convert_type(p & _MHI32, jnp.float32)
    return lo, hi


def _mlp_body(bond_ref, ss_ref, sd_ref, w1a_ref, b1_ref, w2_ref, b2_ref,
              w3_ref, b3_ref, out_ref):
    sl, sh = _unpack_block(ss_ref[...])
    dl, dh = _unpack_block(sd_ref[...])
    s = jnp.concatenate([sl + dl, sh + dh], axis=1)
    h = jnp.dot(bond_ref[...], w1a_ref[...],
                preferred_element_type=jnp.float32)
    h = _leaky(h + s + b1_ref[...])
    h = _leaky(jnp.dot(h.astype(jnp.bfloat16), w2_ref[...],
                       preferred_element_type=jnp.float32) + b2_ref[...])
    h = _leaky(jnp.dot(h.astype(jnp.bfloat16), w3_ref[...],
                       preferred_element_type=jnp.float32) + b3_ref[...])
    out_ref[...] = h


def _edge_mlp(bond_feats, s_src, s_dst, w1a, b1, w2, b2, w3, b3):
    grid = (N_EDGES // BE,)
    bond_feats = bond_feats.astype(jnp.bfloat16)
    w1a = w1a.astype(jnp.bfloat16)
    w2 = w2.astype(jnp.bfloat16)
    w3 = w3.astype(jnp.bfloat16)
    full = lambda shape: pl.BlockSpec(shape, lambda i: (0, 0))
    return pl.pallas_call(
        _mlp_body,
        grid=grid,
        in_specs=[
            pl.BlockSpec((BE, D_EDGE), lambda i: (i, 0)),
            pl.BlockSpec((BE, PK), lambda i: (i, 0)),
            pl.BlockSpec((BE, PK), lambda i: (i, 0)),
            full((D_EDGE, OUT_DIM)),
            full((1, OUT_DIM)),
            full((OUT_DIM, OUT_DIM)),
            full((1, OUT_DIM)),
            full((OUT_DIM, OUT_DIM)),
            full((1, OUT_DIM)),
        ],
        out_specs=pl.BlockSpec((BE, OUT_DIM), lambda i: (i, 0)),
        out_shape=jax.ShapeDtypeStruct((N_EDGES, OUT_DIM), jnp.float32),
    )(bond_feats, s_src, s_dst, w1a, b1, w2, b2, w3, b3)


def kernel(atom_feats, bond_feats, edge_index, W1, b1, W2, b2, W3, b3):
    w1a = W1[:D_EDGE]
    w1b = W1[D_EDGE:]
    src = edge_index[0]
    dst = edge_index[1]
    atom_w = _pack_bf16_pairs(_node_proj(atom_feats, w1b))
    s_src, s_dst = _gather_rows(atom_w, src, dst)
    return _edge_mlp(bond_feats, s_src, s_dst,
                     w1a, b1.reshape(1, -1),
                     W2, b2.reshape(1, -1),
                     W3, b3.reshape(1, -1))


# speedup vs baseline: 1.5364x; 1.5364x over previous
import functools

import jax
import jax.numpy as jnp
import numpy as np
from jax import lax
from jax.experimental import pallas as pl
from jax.experimental.pallas import tpu as pltpu
from jax.experimental.pallas import tpu_sc as plsc

N_NODES = 10000
N_EDGES = 320000
D_FEAT = 128
D_EDGE = 16
OUT_DIM = 128
PK = OUT_DIM // 2

NC = 2
NS = 16
NW = NC * NS
EW = N_EDGES // NW
C = 80
NCHUNK = EW // C
UNROLL = 5
NSUPER = NCHUNK // UNROLL


def _leaky(x):
    return jnp.where(x >= 0, x, 0.01 * x)


def _node_proj_body(atom_ref, w_ref, out_ref):
    out_ref[...] = jnp.dot(atom_ref[...], w_ref[...],
                           preferred_element_type=jnp.float32,
                           precision=lax.Precision.HIGHEST)


def _node_proj(atom_feats, w1b):
    return pl.pallas_call(
        _node_proj_body,
        out_shape=jax.ShapeDtypeStruct((N_NODES, OUT_DIM), jnp.float32),
    )(atom_feats, w1b)


def _pack_bf16_pairs(aw):
    lo = lax.bitcast_convert_type(aw[:, :PK].astype(jnp.bfloat16), jnp.uint16)
    hi = lax.bitcast_convert_type(aw[:, PK:].astype(jnp.bfloat16), jnp.uint16)
    packed = (hi.astype(jnp.uint32) << 16) | lo.astype(jnp.uint32)
    return lax.bitcast_convert_type(packed, jnp.int32)


@functools.partial(
    pl.kernel,
    mesh=plsc.VectorSubcoreMesh(core_axis_name="c", subcore_axis_name="s"),
    out_type=jax.ShapeDtypeStruct((N_EDGES, OUT_DIM), jnp.int32),
    scratch_types=[
        pltpu.VMEM((EW,), jnp.int32),
        pltpu.VMEM((EW,), jnp.int32),
        pltpu.VMEM((2 * UNROLL, C, PK), jnp.int32),
        pltpu.SemaphoreType.DMA,
        pltpu.SemaphoreType.DMA,
    ],
    compiler_params=pltpu.CompilerParams(use_tc_tiling_on_sc=False),
)
def _gather_rows(table, src_hbm, dst_hbm, out_hbm,
                 idx_s, idx_d, rows, sem_g, sem_w):
    wid = lax.axis_index("s") * NC + lax.axis_index("c")
    base = wid * EW
    pltpu.sync_copy(src_hbm.at[pl.ds(base, EW)], idx_s)
    pltpu.sync_copy(dst_hbm.at[pl.ds(base, EW)], idx_d)

    def super_chunk(t, carry):
        j0 = t * UNROLL
        gathers = []
        for u in range(UNROLL):
            sl = pl.ds((j0 + u) * C, C)
            c1 = pltpu.async_copy(table.at[idx_s.at[sl]], rows.at[2 * u], sem_g)
            c2 = pltpu.async_copy(table.at[idx_d.at[sl]], rows.at[2 * u + 1],
                                  sem_g)
            gathers.append((c1, c2))
        writebacks = []
        for u in range(UNROLL):
            c1, c2 = gathers[u]
            c1.wait()
            c2.wait()
            off = base + (j0 + u) * C
            writebacks.append(pltpu.async_copy(
                rows.at[2 * u], out_hbm.at[pl.ds(off, C), pl.ds(0, PK)],
                sem_w))
            writebacks.append(pltpu.async_copy(
                rows.at[2 * u + 1], out_hbm.at[pl.ds(off, C), pl.ds(PK, PK)],
                sem_w))
        for wb in writebacks:
            wb.wait()
        return carry

    lax.fori_loop(0, NSUPER, super_chunk, 0)


BE = 2000

_MHI32 = np.int32(-65536)


def _mlp_body(bond_ref, s_ref, w1a_ref, b1_ref, w2_ref, b2_ref,
              w3_ref, b3_ref, out_ref):
    p = s_ref[...]
    lo = lax.bitcast_convert_type(p << 16, jnp.float32)
    hi = lax.bitcast_convert_type(p & _MHI32, jnp.float32)
    s = jnp.concatenate([lo[:, :PK] + lo[:, PK:], hi[:, :PK] + hi[:, PK:]],
                        axis=1)
    h = jnp.dot(bond_ref[...], w1a_ref[...],
                preferred_element_type=jnp.float32)
    h = _leaky(h + s + b1_ref[...])
    h = _leaky(jnp.dot(h.astype(jnp.bfloat16), w2_ref[...],
                       preferred_element_type=jnp.float32) + b2_ref[...])
    h = _leaky(jnp.dot(h.astype(jnp.bfloat16), w3_ref[...],
                       preferred_element_type=jnp.float32) + b3_ref[...])
    out_ref[...] = h


def _edge_mlp(bond_feats, s_packed, w1a, b1, w2, b2, w3, b3):
    grid = (N_EDGES // BE,)
    bond_feats = bond_feats.astype(jnp.bfloat16)
    w1a = w1a.astype(jnp.bfloat16)
    w2 = w2.astype(jnp.bfloat16)
    w3 = w3.astype(jnp.bfloat16)
    full = lambda shape: pl.BlockSpec(shape, lambda i: (0, 0))
    return pl.pallas_call(
        _mlp_body,
        grid=grid,
        in_specs=[
            pl.BlockSpec((BE, D_EDGE), lambda i: (i, 0)),
            pl.BlockSpec((BE, OUT_DIM), lambda i: (i, 0)),
            full((D_EDGE, OUT_DIM)),
            full((1, OUT_DIM)),
            full((OUT_DIM, OUT_DIM)),
            full((1, OUT_DIM)),
            full((OUT_DIM, OUT_DIM)),
            full((1, OUT_DIM)),
        ],
        out_specs=pl.BlockSpec((BE, OUT_DIM), lambda i: (i, 0)),
        out_shape=jax.ShapeDtypeStruct((N_EDGES, OUT_DIM), jnp.float32),
    )(bond_feats, s_packed, w1a, b1, w2, b2, w3, b3)


def kernel(atom_feats, bond_feats, edge_index, W1, b1, W2, b2, W3, b3):
    w1a = W1[:D_EDGE]
    w1b = W1[D_EDGE:]
    src = edge_index[0]
    dst = edge_index[1]
    atom_w = _pack_bf16_pairs(_node_proj(atom_feats, w1b))
    s_packed = _gather_rows(atom_w, src, dst)
    return _edge_mlp(bond_feats, s_packed,
                     w1a, b1.reshape(1, -1),
                     W2, b2.reshape(1, -1),
                     W3, b3.reshape(1, -1))


# speedup vs baseline: 1.6372x; 1.0656x over previous
import functools

import jax
import jax.numpy as jnp
import numpy as np
from jax import lax
from jax.experimental import pallas as pl
from jax.experimental.pallas import tpu as pltpu
from jax.experimental.pallas import tpu_sc as plsc

N_NODES = 10000
N_EDGES = 320000
D_FEAT = 128
D_EDGE = 16
OUT_DIM = 128
PK = OUT_DIM // 2

NSPLIT = 2
ESPLIT = N_EDGES // NSPLIT

NC = 2
NS = 16
NW = NC * NS
EW = ESPLIT // NW
C = 40
NCHUNK = EW // C
UNROLL = 5
NSUPER = NCHUNK // UNROLL


def _leaky(x):
    return jnp.where(x >= 0, x, 0.01 * x)


def _node_proj_body(atom_ref, w_ref, out_ref):
    out_ref[...] = jnp.dot(atom_ref[...], w_ref[...],
                           preferred_element_type=jnp.float32,
                           precision=lax.Precision.HIGHEST)


def _node_proj(atom_feats, w1b):
    return pl.pallas_call(
        _node_proj_body,
        out_shape=jax.ShapeDtypeStruct((N_NODES, OUT_DIM), jnp.float32),
    )(atom_feats, w1b)


def _pack_bf16_pairs(aw):
    lo = lax.bitcast_convert_type(aw[:, :PK].astype(jnp.bfloat16), jnp.uint16)
    hi = lax.bitcast_convert_type(aw[:, PK:].astype(jnp.bfloat16), jnp.uint16)
    packed = (hi.astype(jnp.uint32) << 16) | lo.astype(jnp.uint32)
    return lax.bitcast_convert_type(packed, jnp.int32)


def _make_gather(split):
    @functools.partial(
        pl.kernel,
        mesh=plsc.VectorSubcoreMesh(core_axis_name="c", subcore_axis_name="s"),
        out_type=jax.ShapeDtypeStruct((ESPLIT, OUT_DIM), jnp.int32),
        scratch_types=[
            pltpu.VMEM((EW,), jnp.int32),
            pltpu.VMEM((EW,), jnp.int32),
            pltpu.VMEM((2 * UNROLL, C, PK), jnp.int32),
            pltpu.SemaphoreType.DMA,
            pltpu.SemaphoreType.DMA,
        ],
        compiler_params=pltpu.CompilerParams(use_tc_tiling_on_sc=False),
    )
    def _gather_rows(table, src_hbm, dst_hbm, out_hbm,
                     idx_s, idx_d, rows, sem_g, sem_w):
        wid = lax.axis_index("s") * NC + lax.axis_index("c")
        base_in = split * ESPLIT + wid * EW
        base_out = wid * EW
        pltpu.sync_copy(src_hbm.at[pl.ds(base_in, EW)], idx_s)
        pltpu.sync_copy(dst_hbm.at[pl.ds(base_in, EW)], idx_d)

        def super_chunk(t, carry):
            j0 = t * UNROLL
            gathers = []
            for u in range(UNROLL):
                sl = pl.ds((j0 + u) * C, C)
                c1 = pltpu.async_copy(table.at[idx_s.at[sl]], rows.at[2 * u],
                                      sem_g)
                c2 = pltpu.async_copy(table.at[idx_d.at[sl]],
                                      rows.at[2 * u + 1], sem_g)
                gathers.append((c1, c2))
            writebacks = []
            for u in range(UNROLL):
                c1, c2 = gathers[u]
                c1.wait()
                c2.wait()
                off = base_out + (j0 + u) * C
                writebacks.append(pltpu.async_copy(
                    rows.at[2 * u], out_hbm.at[pl.ds(off, C), pl.ds(0, PK)],
                    sem_w))
                writebacks.append(pltpu.async_copy(
                    rows.at[2 * u + 1],
                    out_hbm.at[pl.ds(off, C), pl.ds(PK, PK)], sem_w))
            for wb in writebacks:
                wb.wait()
            return carry

        lax.fori_loop(0, NSUPER, super_chunk, 0)

    return _gather_rows


_gather_split0 = _make_gather(0)
_gather_split1 = _make_gather(1)


BE = 2000
NBLK = ESPLIT // BE

_MHI32 = np.int32(-65536)


def _mlp_body(bond_ref, s_ref, w1a_ref, b1_ref, w2_ref, b2_ref,
              w3_ref, b3_ref, out_ref):
    p = s_ref[...]
    lo = lax.bitcast_convert_type(p << 16, jnp.float32)
    hi = lax.bitcast_convert_type(p & _MHI32, jnp.float32)
    s = jnp.concatenate([lo[:, :PK] + lo[:, PK:], hi[:, :PK] + hi[:, PK:]],
                        axis=1)
    h = jnp.dot(bond_ref[...], w1a_ref[...],
                preferred_element_type=jnp.float32)
    h = _leaky(h + s + b1_ref[...])
    h = _leaky(jnp.dot(h.astype(jnp.bfloat16), w2_ref[...],
                       preferred_element_type=jnp.float32) + b2_ref[...])
    h = _leaky(jnp.dot(h.astype(jnp.bfloat16), w3_ref[...],
                       preferred_element_type=jnp.float32) + b3_ref[...])
    out_ref[...] = h


def _mlp_body_aliased(bond_ref, s_ref, w1a_ref, b1_ref, w2_ref, b2_ref,
                      w3_ref, b3_ref, prev_ref, out_ref):
    del prev_ref
    _mlp_body(bond_ref, s_ref, w1a_ref, b1_ref, w2_ref, b2_ref,
              w3_ref, b3_ref, out_ref)


def _edge_mlp(split, bond_bf16, s_packed, w1a, b1, w2, b2, w3, b3,
              prev_out=None):
    grid = (NBLK,)
    full = lambda shape: pl.BlockSpec(shape, lambda i: (0, 0))
    in_specs = [
        pl.BlockSpec((BE, D_EDGE), lambda i: (split * NBLK + i, 0)),
        pl.BlockSpec((BE, OUT_DIM), lambda i: (i, 0)),
        full((D_EDGE, OUT_DIM)),
        full((1, OUT_DIM)),
        full((OUT_DIM, OUT_DIM)),
        full((1, OUT_DIM)),
        full((OUT_DIM, OUT_DIM)),
        full((1, OUT_DIM)),
    ]
    args = [bond_bf16, s_packed, w1a, b1, w2, b2, w3, b3]
    body = _mlp_body
    aliases = {}
    if prev_out is not None:
        in_specs.append(pl.BlockSpec((8, OUT_DIM), lambda i: (0, 0)))
        args.append(prev_out)
        body = _mlp_body_aliased
        aliases = {8: 0}
    return pl.pallas_call(
        body,
        grid=grid,
        in_specs=in_specs,
        out_specs=pl.BlockSpec((BE, OUT_DIM), lambda i: (split * NBLK + i, 0)),
        out_shape=jax.ShapeDtypeStruct((N_EDGES, OUT_DIM), jnp.float32),
        input_output_aliases=aliases,
    )(*args)


def kernel(atom_feats, bond_feats, edge_index, W1, b1, W2, b2, W3, b3):
    w1a = W1[:D_EDGE].astype(jnp.bfloat16)
    w2 = W2.astype(jnp.bfloat16)
    w3 = W3.astype(jnp.bfloat16)
    b1r = b1.reshape(1, -1)
    b2r = b2.reshape(1, -1)
    b3r = b3.reshape(1, -1)
    bond_bf16 = bond_feats.astype(jnp.bfloat16)
    src = edge_index[0]
    dst = edge_index[1]
    atom_w = _pack_bf16_pairs(_node_proj(atom_feats, W1[D_EDGE:]))
    s0 = _gather_split0(atom_w, src, dst)
    s1 = _gather_split1(atom_w, src, dst)
    out = _edge_mlp(0, bond_bf16, s0, w1a, b1r, w2, b2r, w3, b3r)
    out = _edge_mlp(1, bond_bf16, s1, w1a, b1r, w2, b2r, w3, b3r,
                    prev_out=out)
    return out


# speedup vs baseline: 1.9051x; 1.1636x over previous
import functools

import jax
import jax.numpy as jnp
import numpy as np
from jax import lax
from jax.experimental import pallas as pl
from jax.experimental.pallas import tpu as pltpu
from jax.experimental.pallas import tpu_sc as plsc

N_NODES = 10000
N_EDGES = 320000
D_FEAT = 128
D_EDGE = 16
OUT_DIM = 128
PK = OUT_DIM // 2

NSPLIT = 2
ESPLIT = N_EDGES // NSPLIT

NC = 2
NS = 16
NW = NC * NS
EW = ESPLIT // NW
C = 40
NCHUNK = EW // C
UNROLL = 5
NSUPER = NCHUNK // UNROLL


def _leaky(x):
    return jnp.where(x >= 0, x, 0.01 * x)


def _node_proj_body(atom_ref, w_ref, out_ref):
    out_ref[...] = jnp.dot(atom_ref[...], w_ref[...],
                           preferred_element_type=jnp.float32,
                           precision=lax.Precision.HIGHEST)


def _node_proj(atom_feats, w1b):
    return pl.pallas_call(
        _node_proj_body,
        out_shape=jax.ShapeDtypeStruct((N_NODES, OUT_DIM), jnp.float32),
    )(atom_feats, w1b)


def _pack_bf16_pairs(aw):
    lo = lax.bitcast_convert_type(aw[:, :PK].astype(jnp.bfloat16), jnp.uint16)
    hi = lax.bitcast_convert_type(aw[:, PK:].astype(jnp.bfloat16), jnp.uint16)
    packed = (hi.astype(jnp.uint32) << 16) | lo.astype(jnp.uint32)
    return lax.bitcast_convert_type(packed, jnp.int32)


def _make_gather(split):
    @functools.partial(
        pl.kernel,
        mesh=plsc.VectorSubcoreMesh(core_axis_name="c", subcore_axis_name="s"),
        out_type=jax.ShapeDtypeStruct((ESPLIT, OUT_DIM), jnp.int32),
        scratch_types=[
            pltpu.VMEM((EW,), jnp.int32),
            pltpu.VMEM((EW,), jnp.int32),
            pltpu.VMEM((2 * UNROLL, C, PK), jnp.int32),
            pltpu.SemaphoreType.DMA,
            pltpu.SemaphoreType.DMA,
        ],
        compiler_params=pltpu.CompilerParams(use_tc_tiling_on_sc=False),
    )
    def _gather_rows(table, src_hbm, dst_hbm, out_hbm,
                     idx_s, idx_d, rows, sem_g, sem_w):
        wid = lax.axis_index("s") * NC + lax.axis_index("c")
        base_in = split * ESPLIT + wid * EW
        base_out = wid * EW
        pltpu.sync_copy(src_hbm.at[pl.ds(base_in, EW)], idx_s)
        pltpu.sync_copy(dst_hbm.at[pl.ds(base_in, EW)], idx_d)

        def super_chunk(t, carry):
            j0 = t * UNROLL
            gathers = []
            for u in range(UNROLL):
                sl = pl.ds((j0 + u) * C, C)
                c1 = pltpu.async_copy(table.at[idx_s.at[sl]], rows.at[2 * u],
                                      sem_g)
                c2 = pltpu.async_copy(table.at[idx_d.at[sl]],
                                      rows.at[2 * u + 1], sem_g)
                gathers.append((c1, c2))
            writebacks = []
            for u in range(UNROLL):
                c1, c2 = gathers[u]
                c1.wait()
                c2.wait()
                off = base_out + (j0 + u) * C
                writebacks.append(pltpu.async_copy(
                    rows.at[2 * u], out_hbm.at[pl.ds(off, C), pl.ds(0, PK)],
                    sem_w))
                writebacks.append(pltpu.async_copy(
                    rows.at[2 * u + 1],
                    out_hbm.at[pl.ds(off, C), pl.ds(PK, PK)], sem_w))
            for wb in writebacks:
                wb.wait()
            return carry

        lax.fori_loop(0, NSUPER, super_chunk, 0)

    return _gather_rows


_gather_split0 = _make_gather(0)
_gather_split1 = _make_gather(1)


BE = 4000
NBLK = ESPLIT // BE

_MHI32 = np.int32(-65536)


def _mlp_body(bond_ref, s_ref, w1a_ref, b1_ref, w2_ref, b2_ref,
              w3_ref, b3_ref, out_ref):
    p = s_ref[...]
    lo = lax.bitcast_convert_type(p << 16, jnp.float32)
    hi = lax.bitcast_convert_type(p & _MHI32, jnp.float32)
    s = jnp.concatenate([lo[:, :PK] + lo[:, PK:], hi[:, :PK] + hi[:, PK:]],
                        axis=1)
    h = jnp.dot(bond_ref[...], w1a_ref[...],
                preferred_element_type=jnp.float32)
    h = _leaky(h + s + b1_ref[...])
    h = _leaky(jnp.dot(h.astype(jnp.bfloat16), w2_ref[...],
                       preferred_element_type=jnp.float32) + b2_ref[...])
    h = _leaky(jnp.dot(h.astype(jnp.bfloat16), w3_ref[...],
                       preferred_element_type=jnp.float32) + b3_ref[...])
    out_ref[...] = h


def _mlp_body_aliased(bond_ref, s_ref, w1a_ref, b1_ref, w2_ref, b2_ref,
                      w3_ref, b3_ref, prev_ref, out_ref):
    del prev_ref
    _mlp_body(bond_ref, s_ref, w1a_ref, b1_ref, w2_ref, b2_ref,
              w3_ref, b3_ref, out_ref)


def _edge_mlp(split, bond_bf16, s_packed, w1a, b1, w2, b2, w3, b3,
              prev_out=None):
    grid = (NBLK,)
    full = lambda shape: pl.BlockSpec(shape, lambda i: (0, 0))
    in_specs = [
        pl.BlockSpec((BE, D_EDGE), lambda i: (split * NBLK + i, 0)),
        pl.BlockSpec((BE, OUT_DIM), lambda i: (i, 0)),
        full((D_EDGE, OUT_DIM)),
        full((1, OUT_DIM)),
        full((OUT_DIM, OUT_DIM)),
        full((1, OUT_DIM)),
        full((OUT_DIM, OUT_DIM)),
        full((1, OUT_DIM)),
    ]
    args = [bond_bf16, s_packed, w1a, b1, w2, b2, w3, b3]
    body = _mlp_body
    aliases = {}
    if prev_out is not None:
        in_specs.append(pl.BlockSpec((8, OUT_DIM), lambda i: (0, 0)))
        args.append(prev_out)
        body = _mlp_body_aliased
        aliases = {8: 0}
    return pl.pallas_call(
        body,
        grid=grid,
        in_specs=in_specs,
        out_specs=pl.BlockSpec((BE, OUT_DIM), lambda i: (split * NBLK + i, 0)),
        out_shape=jax.ShapeDtypeStruct((N_EDGES, OUT_DIM), jnp.float32),
        input_output_aliases=aliases,
    )(*args)


def kernel(atom_feats, bond_feats, edge_index, W1, b1, W2, b2, W3, b3):
    w1a = W1[:D_EDGE].astype(jnp.bfloat16)
    w2 = W2.astype(jnp.bfloat16)
    w3 = W3.astype(jnp.bfloat16)
    b1r = b1.reshape(1, -1)
    b2r = b2.reshape(1, -1)
    b3r = b3.reshape(1, -1)
    bond_bf16 = bond_feats.astype(jnp.bfloat16)
    src = edge_index[0]
    dst = edge_index[1]
    atom_w = _pack_bf16_pairs(_node_proj(atom_feats, W1[D_EDGE:]))
    s0 = _gather_split0(atom_w, src, dst)
    s1 = _gather_split1(atom_w, src, dst)
    out = _edge_mlp(0, bond_bf16, s0, w1a, b1r, w2, b2r, w3, b3r)
    out = _edge_mlp(1, bond_bf16, s1, w1a, b1r, w2, b2r, w3, b3r,
                    prev_out=out)
    return out


# speedup vs baseline: 1.9941x; 1.0467x over previous
import functools

import jax
import jax.numpy as jnp
import numpy as np
from jax import lax
from jax.experimental import pallas as pl
from jax.experimental.pallas import tpu as pltpu
from jax.experimental.pallas import tpu_sc as plsc

N_NODES = 10000
N_EDGES = 320000
D_FEAT = 128
D_EDGE = 16
OUT_DIM = 128
PK = OUT_DIM // 2

NSPLIT = 2
ESPLIT = N_EDGES // NSPLIT

NC = 2
NS = 16
NW = NC * NS
EW = ESPLIT // NW
C = 40
NCHUNK = EW // C
UNROLL = 5
NSUPER = NCHUNK // UNROLL


def _leaky(x):
    return jnp.where(x >= 0, x, 0.01 * x)


def _node_proj_body(atom_ref, w_ref, out_ref):
    out_ref[...] = jnp.dot(atom_ref[...], w_ref[...],
                           preferred_element_type=jnp.float32,
                           precision=lax.Precision.HIGHEST)


def _node_proj(atom_feats, w1b):
    return pl.pallas_call(
        _node_proj_body,
        out_shape=jax.ShapeDtypeStruct((N_NODES, OUT_DIM), jnp.float32),
    )(atom_feats, w1b)


def _pack_bf16_pairs(aw):
    lo = lax.bitcast_convert_type(aw[:, :PK].astype(jnp.bfloat16), jnp.uint16)
    hi = lax.bitcast_convert_type(aw[:, PK:].astype(jnp.bfloat16), jnp.uint16)
    packed = (hi.astype(jnp.uint32) << 16) | lo.astype(jnp.uint32)
    return lax.bitcast_convert_type(packed, jnp.int32)


def _make_gather(split):
    @functools.partial(
        pl.kernel,
        mesh=plsc.VectorSubcoreMesh(core_axis_name="c", subcore_axis_name="s"),
        out_type=jax.ShapeDtypeStruct((ESPLIT, OUT_DIM), jnp.int32),
        scratch_types=[
            pltpu.VMEM((EW,), jnp.int32),
            pltpu.VMEM((EW,), jnp.int32),
            pltpu.VMEM((2 * UNROLL, C, PK), jnp.int32),
            pltpu.SemaphoreType.DMA,
            pltpu.SemaphoreType.DMA,
        ],
        compiler_params=pltpu.CompilerParams(use_tc_tiling_on_sc=False),
    )
    def _gather_rows(table, src_hbm, dst_hbm, out_hbm,
                     idx_s, idx_d, rows, sem_g, sem_w):
        wid = lax.axis_index("s") * NC + lax.axis_index("c")
        base_in = split * ESPLIT + wid * EW
        base_out = wid * EW
        pltpu.sync_copy(src_hbm.at[pl.ds(base_in, EW)], idx_s)
        pltpu.sync_copy(dst_hbm.at[pl.ds(base_in, EW)], idx_d)

        def super_chunk(t, carry):
            j0 = t * UNROLL
            gathers = []
            for u in range(UNROLL):
                sl = pl.ds((j0 + u) * C, C)
                c1 = pltpu.async_copy(table.at[idx_s.at[sl]], rows.at[2 * u],
                                      sem_g)
                c2 = pltpu.async_copy(table.at[idx_d.at[sl]],
                                      rows.at[2 * u + 1], sem_g)
                gathers.append((c1, c2))
            writebacks = []
            for u in range(UNROLL):
                c1, c2 = gathers[u]
                c1.wait()
                c2.wait()
                off = base_out + (j0 + u) * C
                writebacks.append(pltpu.async_copy(
                    rows.at[2 * u], out_hbm.at[pl.ds(off, C), pl.ds(0, PK)],
                    sem_w))
                writebacks.append(pltpu.async_copy(
                    rows.at[2 * u + 1],
                    out_hbm.at[pl.ds(off, C), pl.ds(PK, PK)], sem_w))
            for wb in writebacks:
                wb.wait()
            return carry

        lax.fori_loop(0, NSUPER, super_chunk, 0)

    return _gather_rows


_gather_split0 = _make_gather(0)
_gather_split1 = _make_gather(1)


BE = 8000
NBLK = ESPLIT // BE

_MHI32 = np.int32(-65536)


def _mlp_body(bond_ref, s_ref, w1a_ref, b1_ref, w2_ref, b2_ref,
              w3_ref, b3_ref, out_ref):
    p = s_ref[...]
    lo = lax.bitcast_convert_type(p << 16, jnp.float32)
    hi = lax.bitcast_convert_type(p & _MHI32, jnp.float32)
    s = jnp.concatenate([lo[:, :PK] + lo[:, PK:], hi[:, :PK] + hi[:, PK:]],
                        axis=1)
    h = jnp.dot(bond_ref[...], w1a_ref[...],
                preferred_element_type=jnp.float32)
    h = _leaky(h + s + b1_ref[...])
    h = _leaky(jnp.dot(h.astype(jnp.bfloat16), w2_ref[...],
                       preferred_element_type=jnp.float32) + b2_ref[...])
    h = _leaky(jnp.dot(h.astype(jnp.bfloat16), w3_ref[...],
                       preferred_element_type=jnp.float32) + b3_ref[...])
    out_ref[...] = h


def _mlp_body_aliased(bond_ref, s_ref, w1a_ref, b1_ref, w2_ref, b2_ref,
                      w3_ref, b3_ref, prev_ref, out_ref):
    del prev_ref
    _mlp_body(bond_ref, s_ref, w1a_ref, b1_ref, w2_ref, b2_ref,
              w3_ref, b3_ref, out_ref)


def _edge_mlp(split, bond_bf16, s_packed, w1a, b1, w2, b2, w3, b3,
              prev_out=None):
    grid = (NBLK,)
    full = lambda shape: pl.BlockSpec(shape, lambda i: (0, 0))
    in_specs = [
        pl.BlockSpec((BE, D_EDGE), lambda i: (split * NBLK + i, 0)),
        pl.BlockSpec((BE, OUT_DIM), lambda i: (i, 0)),
        full((D_EDGE, OUT_DIM)),
        full((1, OUT_DIM)),
        full((OUT_DIM, OUT_DIM)),
        full((1, OUT_DIM)),
        full((OUT_DIM, OUT_DIM)),
        full((1, OUT_DIM)),
    ]
    args = [bond_bf16, s_packed, w1a, b1, w2, b2, w3, b3]
    body = _mlp_body
    aliases = {}
    if prev_out is not None:
        in_specs.append(pl.BlockSpec((8, OUT_DIM), lambda i: (0, 0)))
        args.append(prev_out)
        body = _mlp_body_aliased
        aliases = {8: 0}
    return pl.pallas_call(
        body,
        grid=grid,
        in_specs=in_specs,
        out_specs=pl.BlockSpec((BE, OUT_DIM), lambda i: (split * NBLK + i, 0)),
        out_shape=jax.ShapeDtypeStruct((N_EDGES, OUT_DIM), jnp.float32),
        input_output_aliases=aliases,
    )(*args)


def kernel(atom_feats, bond_feats, edge_index, W1, b1, W2, b2, W3, b3):
    w1a = W1[:D_EDGE].astype(jnp.bfloat16)
    w2 = W2.astype(jnp.bfloat16)
    w3 = W3.astype(jnp.bfloat16)
    b1r = b1.reshape(1, -1)
    b2r = b2.reshape(1, -1)
    b3r = b3.reshape(1, -1)
    bond_bf16 = bond_feats.astype(jnp.bfloat16)
    src = edge_index[0]
    dst = edge_index[1]
    atom_w = _pack_bf16_pairs(_node_proj(atom_feats, W1[D_EDGE:]))
    s0 = _gather_split0(atom_w, src, dst)
    s1 = _gather_split1(atom_w, src, dst)
    out = _edge_mlp(0, bond_bf16, s0, w1a, b1r, w2, b2r, w3, b3r)
    out = _edge_mlp(1, bond_bf16, s1, w1a, b1r, w2, b2r, w3, b3r,
                    prev_out=out)
    return out
